# baseline (speedup 1.0000x reference)
import functools

import jax
import jax.numpy as jnp
from jax import lax
from jax.experimental import pallas as pl
from jax.experimental.pallas import tpu as pltpu
from jax.experimental.pallas import tpu_sc as plsc

HIGHEST = lax.Precision.HIGHEST

BLK = 256
NC, NS = 2, 16
NW = NC * NS


def _fiota(shape, dim):
    return lax.broadcasted_iota(jnp.int32, shape, dim).astype(jnp.float32)


def _gate_kernel(x_ref, gw_ref, out_ref, xc_ref):
    x = x_ref[...]
    gw = gw_ref[...]
    xc_ref[...] = x
    logits = lax.dot_general(gw, x, (((1,), (1,)), ((), ())))
    e, n = logits.shape
    eids = _fiota((e, n), 0)
    m0 = jnp.max(logits, axis=0, keepdims=True)
    is0 = (logits == m0)
    i0 = jnp.min(jnp.where(is0, eids, float(e)), axis=0, keepdims=True)
    masked = jnp.where(eids == i0, -jnp.inf, logits)
    m1 = jnp.max(masked, axis=0, keepdims=True)
    is1 = (masked == m1)
    i1 = jnp.min(jnp.where(is1, eids, float(e)), axis=0, keepdims=True)
    w0 = jax.nn.sigmoid(m0 - m1)
    w1 = 1.0 - w0
    out_ref[...] = jnp.concatenate([i0, i1, w0, w1], axis=0)


def _route_kernel(gate_ref, pos_ref, be_ref, live_ref, *, n_tok, n_exp, nb):
    g = gate_ref[...]
    e_s = jnp.concatenate([g[0:1, :], g[1:2, :]], axis=1)
    s = e_s.shape[1]
    eids = _fiota((n_exp, 1), 0)
    onehot = (e_s == eids).astype(jnp.float32)
    ch = 128
    nch = s // ch
    oh3 = onehot.reshape(n_exp, nch, ch)
    r128 = _fiota((ch, ch), 0)
    c128 = _fiota((ch, ch), 1)
    u_incl = (r128 <= c128).astype(jnp.float32)
    incl = lax.dot_general(oh3, u_incl, (((2,), (0,)), ((), ())),
                           precision=HIGHEST)
    tot = jnp.sum(oh3, axis=2)
    rn = _fiota((nch, nch), 0)
    cn = _fiota((nch, nch), 1)
    u_strict = (rn < cn).astype(jnp.float32)
    excl = lax.dot_general(tot, u_strict, (((1,), (0,)), ((), ())),
                           precision=HIGHEST)
    csum = (incl + excl[:, :, None]).reshape(n_exp, s)
    counts = csum[:, s - 1:s]
    rank = jnp.sum((csum - 1.0) * onehot, axis=0, keepdims=True)
    counts_i = counts.astype(jnp.int32)
    padded = ((counts_i + (BLK - 1)) // BLK) * BLK
    padded_f = padded.astype(jnp.float32)
    re_ = _fiota((n_exp, n_exp), 0)
    ce_ = _fiota((n_exp, n_exp), 1)
    l_strict = (ce_ < re_).astype(jnp.float32)
    starts = lax.dot_general(l_strict, padded_f, (((1,), (0,)), ((), ())),
                             precision=HIGHEST)
    pos = jnp.sum(onehot * starts, axis=0, keepdims=True) + rank
    pos_ref[...] = pos.astype(jnp.int32)
    ends = starts + padded_f
    bstart = _fiota((1, nb), 1) * float(BLK)
    fin = jnp.sum((bstart >= ends).astype(jnp.float32), axis=0, keepdims=True)
    be_ref[...] = jnp.minimum(fin, float(n_exp - 1)).astype(jnp.int32)
    real_end = starts + counts
    in_seg = jnp.logical_and(bstart >= starts, bstart < real_end)
    live_ref[...] = jnp.sum(in_seg.astype(jnp.float32), axis=0,
                            keepdims=True).astype(jnp.int32)


def _make_scatter(n_tok, s_total, p_rows):
    rpw = p_rows // NW
    mesh = plsc.VectorSubcoreMesh(core_axis_name="c", subcore_axis_name="s")

    @functools.partial(
        pl.kernel, mesh=mesh,
        compiler_params=pltpu.CompilerParams(needs_layout_passes=False),
        out_type=(jax.ShapeDtypeStruct((p_rows,), jnp.int32),
                  jax.ShapeDtypeStruct((p_rows,), jnp.float32)),
        scratch_types=[
            pltpu.VMEM((s_total,), jnp.int32),
            pltpu.VMEM((s_total,), jnp.float32),
            pltpu.VMEM((rpw,), jnp.int32),
            pltpu.VMEM((rpw,), jnp.float32),
        ],
    )
    def scatter_k(pos_hbm, w_hbm, src_hbm, wout_hbm, pos_v, w_v,
                  src_loc, w_loc):
        wid = lax.axis_index("s") * NC + lax.axis_index("c")
        base = wid * rpw
        pltpu.sync_copy(pos_hbm, pos_v)
        pltpu.sync_copy(w_hbm, w_v)

        def zero_body(i, carry):
            src_loc[pl.ds(i * 16, 16)] = jnp.zeros((16,), jnp.int32)
            w_loc[pl.ds(i * 16, 16)] = jnp.zeros((16,), jnp.float32)
            return carry

        lax.fori_loop(0, rpw // 16, zero_body, 0)
        iota16 = lax.iota(jnp.int32, 16)

        def body(i, carry):
            off = i * 16
            p = pos_v[pl.ds(off, 16)]
            w = w_v[pl.ds(off, 16)]
            svec = off + iota16
            tok = svec - jnp.where(svec >= n_tok, n_tok, 0)
            rel = p - base
            mask = jnp.logical_and(rel >= 0, rel < rpw)
            idx = jnp.where(mask, rel, 0)
            plsc.store_scatter(src_loc, [idx], tok, mask=mask)
            plsc.store_scatter(w_loc, [idx], w, mask=mask)
            return carry

        lax.fori_loop(0, s_total // 16, body, 0)
        pltpu.sync_copy(src_loc, src_hbm.at[pl.ds(base, rpw)])
        pltpu.sync_copy(w_loc, wout_hbm.at[pl.ds(base, rpw)])

    return scatter_k


def _make_gather(n_words, n_rows, dtype):
    rpw = n_rows // NW
    ch = 32 if n_words <= 1024 else 16
    nchunk = rpw // ch
    mesh = plsc.VectorSubcoreMesh(core_axis_name="c", subcore_axis_name="s")

    @functools.partial(
        pl.kernel, mesh=mesh,
        out_type=jax.ShapeDtypeStruct((n_rows, n_words), dtype),
        scratch_types=[
            pltpu.VMEM((rpw,), jnp.int32),
            pltpu.VMEM((2, ch, n_words), dtype),
            pltpu.SemaphoreType.DMA,
            pltpu.SemaphoreType.DMA,
        ],
    )
    def gather_k(x_hbm, idx_hbm, out_hbm, idx_v, rows_v, sem0, sem1):
        wid = lax.axis_index("s") * NC + lax.axis_index("c")
        base = wid * rpw
        pltpu.sync_copy(idx_hbm.at[pl.ds(base, rpw)], idx_v)
        sems = (sem0, sem1)
        descs = [
            pltpu.async_copy(x_hbm.at[idx_v.at[pl.ds(0, ch)]],
                             rows_v.at[0], sems[0]),
            pltpu.async_copy(x_hbm.at[idx_v.at[pl.ds(ch, ch)]],
                             rows_v.at[1], sems[1]),
        ]
        for c in range(nchunk):
            slot = c % 2
            descs[slot].wait()
            pltpu.sync_copy(rows_v.at[slot], out_hbm.at[pl.ds(base + c * ch, ch)])
            nxt = c + 2
            if nxt < nchunk:
                descs[slot] = pltpu.async_copy(
                    x_hbm.at[idx_v.at[pl.ds(nxt * ch, ch)]],
                    rows_v.at[slot], sems[slot])

    return gather_k


def _add_halves_kernel(a_ref, b_ref, o_ref):
    o_ref[...] = a_ref[...] + b_ref[...]


def _stage_a_kernel(be_ref, live_ref, xs_ref, wg_ref, wu_ref, h_ref):
    b = pl.program_id(1)

    @pl.when(live_ref[b] > 0)
    def _():
        xb = xs_ref[...].astype(jnp.bfloat16)
        wg = wg_ref[0].astype(jnp.bfloat16)
        wu = wu_ref[0].astype(jnp.bfloat16)
        g = lax.dot_general(xb, wg, (((1,), (1,)), ((), ())),
                            preferred_element_type=jnp.float32)
        u = lax.dot_general(xb, wu, (((1,), (1,)), ((), ())),
                            preferred_element_type=jnp.float32)
        h = (g * jax.nn.sigmoid(g)) * u
        h_ref[...] = h.astype(jnp.bfloat16)


def _stage_b_kernel(be_ref, live_ref, h_ref, wd_ref, w_ref, y_ref):
    b = pl.program_id(1)

    @pl.when(live_ref[b] > 0)
    def _():
        h = h_ref[...]
        wd = wd_ref[0].astype(jnp.bfloat16)
        y = lax.dot_general(h, wd, (((1,), (1,)), ((), ())),
                            preferred_element_type=jnp.float32)
        w = w_ref[0, 0, :]
        y_ref[...] = y * w[:, None]


def kernel(x, gate_W, Wg, Wu, Wd):
    b_, t_, c_ = x.shape
    n = b_ * t_
    e = gate_W.shape[0]
    hid = Wg.shape[1]
    s = 2 * n
    nb = s // BLK + e
    p_rows = nb * BLK
    ht = 1024
    ct = 1024

    xf = x.reshape(n, c_)

    gblk = 512
    gate_out, x_copy = pl.pallas_call(
        _gate_kernel,
        grid=(n // gblk,),
        in_specs=[pl.BlockSpec((gblk, c_), lambda b: (b, 0)),
                  pl.BlockSpec((e, c_), lambda b: (0, 0))],
        out_specs=(pl.BlockSpec((4, gblk), lambda b: (0, b)),
                   pl.BlockSpec((gblk, c_), lambda b: (b, 0))),
        out_shape=(jax.ShapeDtypeStruct((4, n), jnp.float32),
                   jax.ShapeDtypeStruct((n, c_), jnp.float32)),
    )(xf, gate_W)

    pos_s, be2, live2 = pl.pallas_call(
        functools.partial(_route_kernel, n_tok=n, n_exp=e, nb=nb),
        out_shape=(jax.ShapeDtypeStruct((1, s), jnp.int32),
                   jax.ShapeDtypeStruct((1, nb), jnp.int32),
                   jax.ShapeDtypeStruct((1, nb), jnp.int32)),
    )(gate_out)
    be = be2.reshape(nb)
    live = live2.reshape(nb)

    w_flat = jnp.concatenate([gate_out[2], gate_out[3]])
    src, w_sorted = _make_scatter(n, s, p_rows)(pos_s.reshape(s), w_flat)
    w3 = w_sorted.reshape(nb, 1, BLK)

    xs = _make_gather(c_, p_rows, jnp.float32)(x_copy, src)

    h_sorted = pl.pallas_call(
        _stage_a_kernel,
        grid_spec=pltpu.PrefetchScalarGridSpec(
            num_scalar_prefetch=2,
            grid=(hid // ht, nb),
            in_specs=[
                pl.BlockSpec((BLK, c_), lambda h, b, be, lv: (b, 0)),
                pl.BlockSpec((1, ht, c_), lambda h, b, be, lv: (be[b], h, 0)),
                pl.BlockSpec((1, ht, c_), lambda h, b, be, lv: (be[b], h, 0)),
            ],
            out_specs=pl.BlockSpec((BLK, ht), lambda h, b, be, lv: (b, h)),
        ),
        out_shape=jax.ShapeDtypeStruct((p_rows, hid), jnp.bfloat16),
    )(be, live, xs, Wg, Wu)

    y_sorted = pl.pallas_call(
        _stage_b_kernel,
        grid_spec=pltpu.PrefetchScalarGridSpec(
            num_scalar_prefetch=2,
            grid=(c_ // ct, nb),
            in_specs=[
                pl.BlockSpec((BLK, hid), lambda c, b, be, lv: (b, 0)),
                pl.BlockSpec((1, ct, hid), lambda c, b, be, lv: (be[b], c, 0)),
                pl.BlockSpec((1, 1, BLK), lambda c, b, be, lv: (b, 0, 0)),
            ],
            out_specs=pl.BlockSpec((BLK, ct), lambda c, b, be, lv: (b, c)),
        ),
        out_shape=jax.ShapeDtypeStruct((p_rows, c_), jnp.float32),
    )(be, live, h_sorted, Wd, w3)

    yy = _make_gather(c_, s, jnp.float32)(y_sorted, pos_s.reshape(s))
    ablk = 512
    out = pl.pallas_call(
        _add_halves_kernel,
        grid=(n // ablk,),
        in_specs=[pl.BlockSpec((ablk, c_), lambda b: (b, 0)),
                  pl.BlockSpec((ablk, c_),
                               lambda b, _nb=n // ablk: (b + _nb, 0))],
        out_specs=pl.BlockSpec((ablk, c_), lambda b: (b, 0)),
        out_shape=jax.ShapeDtypeStruct((n, c_), jnp.float32),
    )(yy, yy)
    return out.reshape(b_, t_, c_)


# speedup vs baseline: 1.0263x; 1.0263x over previous
import functools

import jax
import jax.numpy as jnp
from jax import lax
from jax.experimental import pallas as pl
from jax.experimental.pallas import tpu as pltpu
from jax.experimental.pallas import tpu_sc as plsc

HIGHEST = lax.Precision.HIGHEST

BLK = 256
NC, NS = 2, 16
NW = NC * NS


def _fiota(shape, dim):
    return lax.broadcasted_iota(jnp.int32, shape, dim).astype(jnp.float32)


def _gate_kernel(x_ref, gw_ref, out_ref):
    x = x_ref[...]
    gw = gw_ref[...]
    logits = lax.dot_general(gw, x, (((1,), (1,)), ((), ())))
    e, n = logits.shape
    eids = _fiota((e, n), 0)
    m0 = jnp.max(logits, axis=0, keepdims=True)
    is0 = (logits == m0)
    i0 = jnp.min(jnp.where(is0, eids, float(e)), axis=0, keepdims=True)
    masked = jnp.where(eids == i0, -jnp.inf, logits)
    m1 = jnp.max(masked, axis=0, keepdims=True)
    is1 = (masked == m1)
    i1 = jnp.min(jnp.where(is1, eids, float(e)), axis=0, keepdims=True)
    w0 = jax.nn.sigmoid(m0 - m1)
    w1 = 1.0 - w0
    out_ref[...] = jnp.concatenate([i0, i1, w0, w1], axis=0)


def _route_kernel(gate_ref, pos_ref, be_ref, live_ref, *, n_tok, n_exp, nb):
    g = gate_ref[...]
    e_s = jnp.concatenate([g[0:1, :], g[1:2, :]], axis=1)
    s = e_s.shape[1]
    eids = _fiota((n_exp, 1), 0)
    onehot = (e_s == eids).astype(jnp.float32)
    ch = 128
    nch = s // ch
    oh3 = onehot.reshape(n_exp, nch, ch)
    r128 = _fiota((ch, ch), 0)
    c128 = _fiota((ch, ch), 1)
    u_incl = (r128 <= c128).astype(jnp.float32)
    incl = lax.dot_general(oh3, u_incl, (((2,), (0,)), ((), ())),
                           precision=HIGHEST)
    tot = jnp.sum(oh3, axis=2)
    rn = _fiota((nch, nch), 0)
    cn = _fiota((nch, nch), 1)
    u_strict = (rn < cn).astype(jnp.float32)
    excl = lax.dot_general(tot, u_strict, (((1,), (0,)), ((), ())),
                           precision=HIGHEST)
    csum = (incl + excl[:, :, None]).reshape(n_exp, s)
    counts = csum[:, s - 1:s]
    rank = jnp.sum((csum - 1.0) * onehot, axis=0, keepdims=True)
    counts_i = counts.astype(jnp.int32)
    padded = ((counts_i + (BLK - 1)) // BLK) * BLK
    padded_f = padded.astype(jnp.float32)
    re_ = _fiota((n_exp, n_exp), 0)
    ce_ = _fiota((n_exp, n_exp), 1)
    l_strict = (ce_ < re_).astype(jnp.float32)
    starts = lax.dot_general(l_strict, padded_f, (((1,), (0,)), ((), ())),
                             precision=HIGHEST)
    pos = jnp.sum(onehot * starts, axis=0, keepdims=True) + rank
    pos_i = pos.astype(jnp.int32)
    r8 = pos_i & 255
    brev = (((r8 & 1) << 7) | ((r8 & 2) << 5) | ((r8 & 4) << 3)
            | ((r8 & 8) << 1) | ((r8 & 16) >> 1) | ((r8 & 32) >> 3)
            | ((r8 & 64) >> 5) | ((r8 & 128) >> 7))
    pos_ref[...] = (pos_i - r8) | brev
    ends = starts + padded_f
    bstart = _fiota((1, nb), 1) * float(BLK)
    fin = jnp.sum((bstart >= ends).astype(jnp.float32), axis=0, keepdims=True)
    be_ref[...] = jnp.minimum(fin, float(n_exp - 1)).astype(jnp.int32)
    real_end = starts + counts
    in_seg = jnp.logical_and(bstart >= starts, bstart < real_end)
    live_ref[...] = jnp.sum(in_seg.astype(jnp.float32), axis=0,
                            keepdims=True).astype(jnp.int32)


def _make_scatter(n_tok, s_total, p_rows):
    rpw = p_rows // NW
    mesh = plsc.VectorSubcoreMesh(core_axis_name="c", subcore_axis_name="s")

    @functools.partial(
        pl.kernel, mesh=mesh,
        compiler_params=pltpu.CompilerParams(needs_layout_passes=False),
        out_type=(jax.ShapeDtypeStruct((p_rows,), jnp.int32),
                  jax.ShapeDtypeStruct((p_rows,), jnp.float32)),
        scratch_types=[
            pltpu.VMEM((s_total,), jnp.int32),
            pltpu.VMEM((s_total,), jnp.float32),
            pltpu.VMEM((rpw,), jnp.int32),
            pltpu.VMEM((rpw,), jnp.float32),
        ],
    )
    def scatter_k(pos_hbm, w_hbm, src_hbm, wout_hbm, pos_v, w_v,
                  src_loc, w_loc):
        wid = lax.axis_index("s") * NC + lax.axis_index("c")
        base = wid * rpw
        pltpu.sync_copy(pos_hbm, pos_v)
        pltpu.sync_copy(w_hbm, w_v)

        def zero_body(i, carry):
            src_loc[pl.ds(i * 16, 16)] = jnp.zeros((16,), jnp.int32)
            w_loc[pl.ds(i * 16, 16)] = jnp.zeros((16,), jnp.float32)
            return carry

        lax.fori_loop(0, rpw // 16, zero_body, 0)
        iota16 = lax.iota(jnp.int32, 16)

        def body(i, carry):
            off = i * 16
            p = pos_v[pl.ds(off, 16)]
            w = w_v[pl.ds(off, 16)]
            svec = off + iota16
            tok = svec - jnp.where(svec >= n_tok, n_tok, 0)
            rel = p - base
            mask = jnp.logical_and(rel >= 0, rel < rpw)
            idx = jnp.where(mask, rel, 0)
            plsc.store_scatter(src_loc, [idx], tok, mask=mask)
            plsc.store_scatter(w_loc, [idx], w, mask=mask)
            return carry

        lax.fori_loop(0, s_total // 16, body, 0)
        pltpu.sync_copy(src_loc, src_hbm.at[pl.ds(base, rpw)])
        pltpu.sync_copy(w_loc, wout_hbm.at[pl.ds(base, rpw)])

    return scatter_k


def _make_gather(n_words, n_rows, dtype):
    rpw = n_rows // NW
    ch = 32 if n_words <= 1024 else 16
    nchunk = rpw // ch
    mesh = plsc.VectorSubcoreMesh(core_axis_name="c", subcore_axis_name="s")

    @functools.partial(
        pl.kernel, mesh=mesh,
        out_type=jax.ShapeDtypeStruct((n_rows, n_words), dtype),
        scratch_types=[
            pltpu.VMEM((rpw,), jnp.int32),
            pltpu.VMEM((2, ch, n_words), dtype),
            pltpu.SemaphoreType.DMA,
            pltpu.SemaphoreType.DMA,
        ],
    )
    def gather_k(x_hbm, idx_hbm, out_hbm, idx_v, rows_v, sem0, sem1):
        wid = lax.axis_index("s") * NC + lax.axis_index("c")
        base = wid * rpw
        pltpu.sync_copy(idx_hbm.at[pl.ds(base, rpw)], idx_v)
        sems = (sem0, sem1)
        descs = [
            pltpu.async_copy(x_hbm.at[idx_v.at[pl.ds(0, ch)]],
                             rows_v.at[0], sems[0]),
            pltpu.async_copy(x_hbm.at[idx_v.at[pl.ds(ch, ch)]],
                             rows_v.at[1], sems[1]),
        ]
        for c in range(nchunk):
            slot = c % 2
            descs[slot].wait()
            pltpu.sync_copy(rows_v.at[slot], out_hbm.at[pl.ds(base + c * ch, ch)])
            nxt = c + 2
            if nxt < nchunk:
                descs[slot] = pltpu.async_copy(
                    x_hbm.at[idx_v.at[pl.ds(nxt * ch, ch)]],
                    rows_v.at[slot], sems[slot])

    return gather_k


def _add_halves_kernel(a_ref, b_ref, o_ref):
    o_ref[...] = a_ref[...] + b_ref[...]


def _stage_a_kernel(be_ref, live_ref, xs_ref, wg_ref, wu_ref, h_ref):
    b = pl.program_id(1)

    @pl.when(live_ref[b] > 0)
    def _():
        xb = xs_ref[...].astype(jnp.bfloat16)
        wg = wg_ref[0].astype(jnp.bfloat16)
        wu = wu_ref[0].astype(jnp.bfloat16)
        g = lax.dot_general(xb, wg, (((1,), (1,)), ((), ())),
                            preferred_element_type=jnp.float32)
        u = lax.dot_general(xb, wu, (((1,), (1,)), ((), ())),
                            preferred_element_type=jnp.float32)
        h = (g * jax.nn.sigmoid(g)) * u
        h_ref[...] = h.astype(jnp.bfloat16)


def _stage_b_kernel(be_ref, live_ref, h_ref, wd_ref, w_ref, y_ref):
    b = pl.program_id(1)

    @pl.when(live_ref[b] > 0)
    def _():
        h = h_ref[...]
        wd = wd_ref[0].astype(jnp.bfloat16)
        y = lax.dot_general(h, wd, (((1,), (1,)), ((), ())),
                            preferred_element_type=jnp.float32)
        w = w_ref[0, 0, :]
        y_ref[...] = y * w[:, None]


def kernel(x, gate_W, Wg, Wu, Wd):
    b_, t_, c_ = x.shape
    n = b_ * t_
    e = gate_W.shape[0]
    hid = Wg.shape[1]
    s = 2 * n
    nb = s // BLK + e
    p_rows = nb * BLK
    ht = 1024
    ct = 1024

    xf = x.reshape(n, c_)

    gblk = 512
    gate_out = pl.pallas_call(
        _gate_kernel,
        grid=(n // gblk,),
        in_specs=[pl.BlockSpec((gblk, c_), lambda b: (b, 0)),
                  pl.BlockSpec((e, c_), lambda b: (0, 0))],
        out_specs=pl.BlockSpec((4, gblk), lambda b: (0, b)),
        out_shape=jax.ShapeDtypeStruct((4, n), jnp.float32),
    )(xf, gate_W)

    pos_s, be2, live2 = pl.pallas_call(
        functools.partial(_route_kernel, n_tok=n, n_exp=e, nb=nb),
        out_shape=(jax.ShapeDtypeStruct((1, s), jnp.int32),
                   jax.ShapeDtypeStruct((1, nb), jnp.int32),
                   jax.ShapeDtypeStruct((1, nb), jnp.int32)),
    )(gate_out)
    be = be2.reshape(nb)
    live = live2.reshape(nb)

    w_flat = jnp.concatenate([gate_out[2], gate_out[3]])
    src, w_sorted = _make_scatter(n, s, p_rows)(pos_s.reshape(s), w_flat)
    w3 = w_sorted.reshape(nb, 1, BLK)

    xs = _make_gather(c_, p_rows, jnp.float32)(xf, src)

    h_sorted = pl.pallas_call(
        _stage_a_kernel,
        grid_spec=pltpu.PrefetchScalarGridSpec(
            num_scalar_prefetch=2,
            grid=(hid // ht, nb),
            in_specs=[
                pl.BlockSpec((BLK, c_), lambda h, b, be, lv: (b, 0)),
                pl.BlockSpec((1, ht, c_), lambda h, b, be, lv: (be[b], h, 0)),
                pl.BlockSpec((1, ht, c_), lambda h, b, be, lv: (be[b], h, 0)),
            ],
            out_specs=pl.BlockSpec((BLK, ht), lambda h, b, be, lv: (b, h)),
        ),
        out_shape=jax.ShapeDtypeStruct((p_rows, hid), jnp.bfloat16),
    )(be, live, xs, Wg, Wu)

    y_sorted = pl.pallas_call(
        _stage_b_kernel,
        grid_spec=pltpu.PrefetchScalarGridSpec(
            num_scalar_prefetch=2,
            grid=(c_ // ct, nb),
            in_specs=[
                pl.BlockSpec((BLK, hid), lambda c, b, be, lv: (b, 0)),
                pl.BlockSpec((1, ct, hid), lambda c, b, be, lv: (be[b], c, 0)),
                pl.BlockSpec((1, 1, BLK), lambda c, b, be, lv: (b, 0, 0)),
            ],
            out_specs=pl.BlockSpec((BLK, ct), lambda c, b, be, lv: (b, c)),
        ),
        out_shape=jax.ShapeDtypeStruct((p_rows, c_), jnp.float32),
    )(be, live, h_sorted, Wd, w3)

    yy = _make_gather(c_, s, jnp.float32)(y_sorted, pos_s.reshape(s))
    ablk = 512
    out = pl.pallas_call(
        _add_halves_kernel,
        grid=(n // ablk,),
        in_specs=[pl.BlockSpec((ablk, c_), lambda b: (b, 0)),
                  pl.BlockSpec((ablk, c_),
                               lambda b, _nb=n // ablk: (b + _nb, 0))],
        out_specs=pl.BlockSpec((ablk, c_), lambda b: (b, 0)),
        out_shape=jax.ShapeDtypeStruct((n, c_), jnp.float32),
    )(yy, yy)
    return out.reshape(b_, t_, c_)


# speedup vs baseline: 1.2018x; 1.1710x over previous
import functools

import jax
import jax.numpy as jnp
from jax import lax
from jax.experimental import pallas as pl
from jax.experimental.pallas import tpu as pltpu
from jax.experimental.pallas import tpu_sc as plsc

HIGHEST = lax.Precision.HIGHEST

BLK = 256
NC, NS = 2, 16
NW = NC * NS


def _fiota(shape, dim):
    return lax.broadcasted_iota(jnp.int32, shape, dim).astype(jnp.float32)


def _gate_kernel(x_ref, gw_ref, out_ref):
    x = x_ref[...]
    gw = gw_ref[...]
    logits = lax.dot_general(gw, x, (((1,), (1,)), ((), ())))
    e, n = logits.shape
    eids = _fiota((e, n), 0)
    m0 = jnp.max(logits, axis=0, keepdims=True)
    is0 = (logits == m0)
    i0 = jnp.min(jnp.where(is0, eids, float(e)), axis=0, keepdims=True)
    masked = jnp.where(eids == i0, -jnp.inf, logits)
    m1 = jnp.max(masked, axis=0, keepdims=True)
    is1 = (masked == m1)
    i1 = jnp.min(jnp.where(is1, eids, float(e)), axis=0, keepdims=True)
    w0 = jax.nn.sigmoid(m0 - m1)
    w1 = 1.0 - w0
    out_ref[...] = jnp.concatenate([i0, i1, w0, w1], axis=0)


def _route_kernel(gate_ref, pos_ref, be_ref, live_ref, *, n_tok, n_exp, nb):
    g = gate_ref[...]
    e_s = jnp.concatenate([g[0:1, :], g[1:2, :]], axis=1)
    s = e_s.shape[1]
    eids = _fiota((n_exp, 1), 0)
    onehot = (e_s == eids).astype(jnp.float32)
    ch = 128
    nch = s // ch
    oh3 = onehot.reshape(n_exp, nch, ch)
    r128 = _fiota((ch, ch), 0)
    c128 = _fiota((ch, ch), 1)
    u_incl = (r128 <= c128).astype(jnp.float32)
    incl = lax.dot_general(oh3, u_incl, (((2,), (0,)), ((), ())),
                           precision=HIGHEST)
    tot = jnp.sum(oh3, axis=2)
    rn = _fiota((nch, nch), 0)
    cn = _fiota((nch, nch), 1)
    u_strict = (rn < cn).astype(jnp.float32)
    excl = lax.dot_general(tot, u_strict, (((1,), (0,)), ((), ())),
                           precision=HIGHEST)
    csum = (incl + excl[:, :, None]).reshape(n_exp, s)
    counts = csum[:, s - 1:s]
    rank = jnp.sum((csum - 1.0) * onehot, axis=0, keepdims=True)
    counts_i = counts.astype(jnp.int32)
    padded = ((counts_i + (BLK - 1)) // BLK) * BLK
    padded_f = padded.astype(jnp.float32)
    re_ = _fiota((n_exp, n_exp), 0)
    ce_ = _fiota((n_exp, n_exp), 1)
    l_strict = (ce_ < re_).astype(jnp.float32)
    starts = lax.dot_general(l_strict, padded_f, (((1,), (0,)), ((), ())),
                             precision=HIGHEST)
    pos = jnp.sum(onehot * starts, axis=0, keepdims=True) + rank
    pos_i = pos.astype(jnp.int32)
    r8 = pos_i & 255
    brev = (((r8 & 1) << 7) | ((r8 & 2) << 5) | ((r8 & 4) << 3)
            | ((r8 & 8) << 1) | ((r8 & 16) >> 1) | ((r8 & 32) >> 3)
            | ((r8 & 64) >> 5) | ((r8 & 128) >> 7))
    pos_ref[...] = (pos_i - r8) | brev
    ends = starts + padded_f
    bstart = _fiota((1, nb), 1) * float(BLK)
    fin = jnp.sum((bstart >= ends).astype(jnp.float32), axis=0, keepdims=True)
    be_ref[...] = jnp.minimum(fin, float(n_exp - 1)).astype(jnp.int32)
    real_end = starts + counts
    in_seg = jnp.logical_and(bstart >= starts, bstart < real_end)
    live_ref[...] = jnp.sum(in_seg.astype(jnp.float32), axis=0,
                            keepdims=True).astype(jnp.int32)


def _make_scatter(n_tok, s_total, p_rows):
    rpw = p_rows // NW
    mesh = plsc.VectorSubcoreMesh(core_axis_name="c", subcore_axis_name="s")

    @functools.partial(
        pl.kernel, mesh=mesh,
        compiler_params=pltpu.CompilerParams(needs_layout_passes=False),
        out_type=(jax.ShapeDtypeStruct((p_rows,), jnp.int32),
                  jax.ShapeDtypeStruct((p_rows,), jnp.float32)),
        scratch_types=[
            pltpu.VMEM((s_total,), jnp.int32),
            pltpu.VMEM((s_total,), jnp.float32),
            pltpu.VMEM((rpw,), jnp.int32),
            pltpu.VMEM((rpw,), jnp.float32),
        ],
    )
    def scatter_k(pos_hbm, w_hbm, src_hbm, wout_hbm, pos_v, w_v,
                  src_loc, w_loc):
        wid = lax.axis_index("s") * NC + lax.axis_index("c")
        base = wid * rpw
        pltpu.sync_copy(pos_hbm, pos_v)
        pltpu.sync_copy(w_hbm, w_v)

        def zero_body(i, carry):
            src_loc[pl.ds(i * 16, 16)] = jnp.zeros((16,), jnp.int32)
            w_loc[pl.ds(i * 16, 16)] = jnp.zeros((16,), jnp.float32)
            return carry

        lax.fori_loop(0, rpw // 16, zero_body, 0)
        iota16 = lax.iota(jnp.int32, 16)

        def body(i, carry):
            off = i * 16
            p = pos_v[pl.ds(off, 16)]
            w = w_v[pl.ds(off, 16)]
            svec = off + iota16
            tok = svec - jnp.where(svec >= n_tok, n_tok, 0)
            rel = p - base
            mask = jnp.logical_and(rel >= 0, rel < rpw)
            idx = jnp.where(mask, rel, 0)
            plsc.store_scatter(src_loc, [idx], tok, mask=mask)
            plsc.store_scatter(w_loc, [idx], w, mask=mask)
            return carry

        lax.fori_loop(0, s_total // 16, body, 0)
        pltpu.sync_copy(src_loc, src_hbm.at[pl.ds(base, rpw)])
        pltpu.sync_copy(w_loc, wout_hbm.at[pl.ds(base, rpw)])

    return scatter_k


def _make_gather(n_words, n_rows, dtype):
    rpw = n_rows // NW
    ch = 32 if n_words <= 1024 else 16
    nchunk = rpw // ch
    mesh = plsc.VectorSubcoreMesh(core_axis_name="c", subcore_axis_name="s")

    @functools.partial(
        pl.kernel, mesh=mesh,
        out_type=jax.ShapeDtypeStruct((n_rows, n_words), dtype),
        scratch_types=[
            pltpu.VMEM((rpw,), jnp.int32),
            pltpu.VMEM((2, ch, n_words), dtype),
            pltpu.SemaphoreType.DMA,
            pltpu.SemaphoreType.DMA,
        ],
    )
    def gather_k(x_hbm, idx_hbm, out_hbm, idx_v, rows_v, sem0, sem1):
        wid = lax.axis_index("s") * NC + lax.axis_index("c")
        base = wid * rpw
        pltpu.sync_copy(idx_hbm.at[pl.ds(base, rpw)], idx_v)
        sems = (sem0, sem1)
        descs = [
            pltpu.async_copy(x_hbm.at[idx_v.at[pl.ds(0, ch)]],
                             rows_v.at[0], sems[0]),
            pltpu.async_copy(x_hbm.at[idx_v.at[pl.ds(ch, ch)]],
                             rows_v.at[1], sems[1]),
        ]
        for c in range(nchunk):
            slot = c % 2
            descs[slot].wait()
            pltpu.sync_copy(rows_v.at[slot], out_hbm.at[pl.ds(base + c * ch, ch)])
            nxt = c + 2
            if nxt < nchunk:
                descs[slot] = pltpu.async_copy(
                    x_hbm.at[idx_v.at[pl.ds(nxt * ch, ch)]],
                    rows_v.at[slot], sems[slot])

    return gather_k


def _make_xscatter(n_tok, n_embd, p_rows):
    tpw = n_tok // NW
    ch = 16
    nck = tpw // ch
    mesh = plsc.VectorSubcoreMesh(core_axis_name="c", subcore_axis_name="s")

    @functools.partial(
        pl.kernel, mesh=mesh,
        out_type=jax.ShapeDtypeStruct((p_rows, n_embd), jnp.float32),
        scratch_types=[
            pltpu.VMEM((2 * nck, ch), jnp.int32),
            pltpu.VMEM((2, ch, n_embd), jnp.float32),
            pltpu.SemaphoreType.DMA,
            pltpu.SemaphoreType.DMA,
        ],
    )
    def xscat_k(x_hbm, idx_hbm, out_hbm, idx_v, rows_v, sem0, sem1):
        wid = lax.axis_index("s") * NC + lax.axis_index("c")
        base = wid * tpw
        pltpu.sync_copy(idx_hbm.at[wid], idx_v)
        sems = (sem0, sem1)
        descs = [None, None]
        for c in range(nck):
            slot = c % 2
            if descs[slot] is not None:
                descs[slot][0].wait()
                descs[slot][1].wait()
            pltpu.sync_copy(x_hbm.at[pl.ds(base + c * ch, ch)],
                            rows_v.at[slot])
            d0 = pltpu.async_copy(rows_v.at[slot],
                                  out_hbm.at[idx_v.at[2 * c]], sems[slot])
            d1 = pltpu.async_copy(rows_v.at[slot],
                                  out_hbm.at[idx_v.at[2 * c + 1]], sems[slot])
            descs[slot] = (d0, d1)
        for slot in (0, 1):
            if descs[slot] is not None:
                descs[slot][0].wait()
                descs[slot][1].wait()

    return xscat_k


def _add_halves_kernel(a_ref, b_ref, o_ref):
    o_ref[...] = a_ref[...] + b_ref[...]


def _stage_a_kernel(be_ref, live_ref, xs_ref, wg_ref, wu_ref, h_ref):
    b = pl.program_id(1)

    @pl.when(live_ref[b] > 0)
    def _():
        xb = xs_ref[...].astype(jnp.bfloat16)
        wg = wg_ref[0].astype(jnp.bfloat16)
        wu = wu_ref[0].astype(jnp.bfloat16)
        g = lax.dot_general(xb, wg, (((1,), (1,)), ((), ())),
                            preferred_element_type=jnp.float32)
        u = lax.dot_general(xb, wu, (((1,), (1,)), ((), ())),
                            preferred_element_type=jnp.float32)
        h = (g * jax.nn.sigmoid(g)) * u
        h_ref[...] = h.astype(jnp.bfloat16)


def _stage_b_kernel(be_ref, live_ref, h_ref, wd_ref, w_ref, y_ref):
    b = pl.program_id(1)

    @pl.when(live_ref[b] > 0)
    def _():
        h = h_ref[...]
        wd = wd_ref[0].astype(jnp.bfloat16)
        y = lax.dot_general(h, wd, (((1,), (1,)), ((), ())),
                            preferred_element_type=jnp.float32)
        w = w_ref[0, 0, :]
        y_ref[...] = y * w[:, None]


def kernel(x, gate_W, Wg, Wu, Wd):
    b_, t_, c_ = x.shape
    n = b_ * t_
    e = gate_W.shape[0]
    hid = Wg.shape[1]
    s = 2 * n
    nb = s // BLK + e
    p_rows = nb * BLK
    ht = 1024
    ct = 1024

    xf = x.reshape(n, c_)

    gblk = 512
    gate_out = pl.pallas_call(
        _gate_kernel,
        grid=(n // gblk,),
        in_specs=[pl.BlockSpec((gblk, c_), lambda b: (b, 0)),
                  pl.BlockSpec((e, c_), lambda b: (0, 0))],
        out_specs=pl.BlockSpec((4, gblk), lambda b: (0, b)),
        out_shape=jax.ShapeDtypeStruct((4, n), jnp.float32),
    )(xf, gate_W)

    pos_s, be2, live2 = pl.pallas_call(
        functools.partial(_route_kernel, n_tok=n, n_exp=e, nb=nb),
        out_shape=(jax.ShapeDtypeStruct((1, s), jnp.int32),
                   jax.ShapeDtypeStruct((1, nb), jnp.int32),
                   jax.ShapeDtypeStruct((1, nb), jnp.int32)),
    )(gate_out)
    be = be2.reshape(nb)
    live = live2.reshape(nb)

    w_flat = jnp.concatenate([gate_out[2], gate_out[3]])
    src, w_sorted = _make_scatter(n, s, p_rows)(pos_s.reshape(s), w_flat)
    w3 = w_sorted.reshape(nb, 1, BLK)

    tpw = n // NW
    xck = tpw // 16
    idx3 = (pos_s.reshape(2, NW, xck, 16).transpose(1, 2, 0, 3)
            .reshape(NW, 2 * xck, 16))
    xs = _make_xscatter(n, c_, p_rows)(xf, idx3)

    h_sorted = pl.pallas_call(
        _stage_a_kernel,
        grid_spec=pltpu.PrefetchScalarGridSpec(
            num_scalar_prefetch=2,
            grid=(hid // ht, nb),
            in_specs=[
                pl.BlockSpec((BLK, c_), lambda h, b, be, lv: (b, 0)),
                pl.BlockSpec((1, ht, c_), lambda h, b, be, lv: (be[b], h, 0)),
                pl.BlockSpec((1, ht, c_), lambda h, b, be, lv: (be[b], h, 0)),
            ],
            out_specs=pl.BlockSpec((BLK, ht), lambda h, b, be, lv: (b, h)),
        ),
        out_shape=jax.ShapeDtypeStruct((p_rows, hid), jnp.bfloat16),
    )(be, live, xs, Wg, Wu)

    y_sorted = pl.pallas_call(
        _stage_b_kernel,
        grid_spec=pltpu.PrefetchScalarGridSpec(
            num_scalar_prefetch=2,
            grid=(c_ // ct, nb),
            in_specs=[
                pl.BlockSpec((BLK, hid), lambda c, b, be, lv: (b, 0)),
                pl.BlockSpec((1, ct, hid), lambda c, b, be, lv: (be[b], c, 0)),
                pl.BlockSpec((1, 1, BLK), lambda c, b, be, lv: (b, 0, 0)),
            ],
            out_specs=pl.BlockSpec((BLK, ct), lambda c, b, be, lv: (b, c)),
        ),
        out_shape=jax.ShapeDtypeStruct((p_rows, c_), jnp.float32),
    )(be, live, h_sorted, Wd, w3)

    yy = _make_gather(c_, s, jnp.float32)(y_sorted, pos_s.reshape(s))
    ablk = 512
    out = pl.pallas_call(
        _add_halves_kernel,
        grid=(n // ablk,),
        in_specs=[pl.BlockSpec((ablk, c_), lambda b: (b, 0)),
                  pl.BlockSpec((ablk, c_),
                               lambda b, _nb=n // ablk: (b + _nb, 0))],
        out_specs=pl.BlockSpec((ablk, c_), lambda b: (b, 0)),
        out_shape=jax.ShapeDtypeStruct((n, c_), jnp.float32),
    )(yy, yy)
    return out.reshape(b_, t_, c_)
